# baseline (speedup 1.0000x reference)
import functools

import jax
import jax.numpy as jnp
from jax import lax
from jax.experimental import pallas as pl
from jax.experimental.pallas import tpu as pltpu
from jax.experimental.pallas import tpu_sc as plsc

N = 4096
E = 131072
D1 = 128
D2 = 64

NC = 2
NS = 16
NW = NC * NS
L = 16

CH = 128
EPT = E // NW
NCH = EPT // CH
RPT = N // NS


def _make_spmm(D, with_deg):
  mesh = plsc.VectorSubcoreMesh(core_axis_name="c", subcore_axis_name="s")
  out_type = [jax.ShapeDtypeStruct((NC, N, D), jnp.float32)]
  if with_deg:
    out_type.append(jax.ShapeDtypeStruct((NC, N), jnp.float32))
  scratch = [
      pltpu.VMEM((NCH, CH), jnp.int32),
      pltpu.VMEM((NCH, CH), jnp.int32),
      pltpu.VMEM((NCH, CH), jnp.float32),
      pltpu.VMEM((1, L, CH), jnp.float32),
      pltpu.VMEM((1, L, CH), jnp.float32),
      pltpu.VMEM((CH, D), jnp.float32),
      pltpu.VMEM((CH, D), jnp.float32),
      pltpu.VMEM_SHARED((N, D), jnp.float32),
      pltpu.VMEM_SHARED((N,), jnp.float32),
      pltpu.SemaphoreType.DMA,
      pltpu.SemaphoreType.DMA,
      pltpu.SemaphoreType.DMA,
      pltpu.SemaphoreType.DMA,
  ]

  @functools.partial(pl.kernel, mesh=mesh, out_type=out_type,
                     scratch_types=scratch)
  def spmm(rows_hbm, cols_hbm, ew_hbm, valx_hbm, xw_hbm, *out_and_scratch):
    if with_deg:
      out_hbm, deg_hbm = out_and_scratch[0], out_and_scratch[1]
      rest = out_and_scratch[2:]
    else:
      out_hbm = out_and_scratch[0]
      rest = out_and_scratch[1:]
    rbuf, cbuf, wbuf, v0, v1, g0, g1, h_sh, deg_sh, sg0, sg1, ss0, ss1 = rest
    cid = lax.axis_index("c")
    sid = lax.axis_index("s")
    wid = sid * NC + cid

    @plsc.parallel_loop(0, CH, unroll=4)
    def _(e):
      for c0 in range(0, D, L):
        g0[e, pl.ds(c0, L)] = jnp.zeros((L,), jnp.float32)

    for k in range(RPT // CH):
      pltpu.sync_copy(g0, h_sh.at[pl.ds(sid * RPT + k * CH, CH)])
    if with_deg:
      for k in range(RPT // D):
        pltpu.sync_copy(g0.at[0], deg_sh.at[pl.ds(sid * RPT + k * D, D)])

    pltpu.sync_copy(rows_hbm.at[pl.ds(wid * NCH, NCH)], rbuf)
    pltpu.sync_copy(cols_hbm.at[pl.ds(wid * NCH, NCH)], cbuf)
    pltpu.sync_copy(ew_hbm.at[pl.ds(wid * NCH, NCH)], wbuf)

    plsc.subcore_barrier()

    def start_fetch(g, v, sg, j):
      pltpu.async_copy(xw_hbm.at[cbuf.at[j]], g, sg)
      pltpu.async_copy(valx_hbm.at[pl.ds(wid * NCH + j, 1)], v, sg)

    def wait_fetch(g, v, sg, j):
      pltpu.make_async_copy(xw_hbm.at[cbuf.at[j]], g, sg).wait()
      pltpu.make_async_copy(valx_hbm.at[pl.ds(wid * NCH + j, 1)], v, sg).wait()

    def scale(g, v):
      @plsc.parallel_loop(0, L, unroll=8)
      def _(e8):
        for jj in range(CH // L):
          e = e8 * (CH // L) + jj
          vv = v[0, e8, pl.ds(jj * L, L)]
          for c0 in range(0, D, L):
            g[e, pl.ds(c0, L)] = g[e, pl.ds(c0, L)] * vv

    def deg_scatter(j):
      if with_deg:
        pltpu.sync_copy(wbuf.at[j], deg_sh.at[rbuf.at[j]], add=True)

    start_fetch(g0, v0, sg0, 0)

    @pl.loop(0, NCH // 2)
    def _(jj):
      j0 = jj * 2
      j1 = j0 + 1
      start_fetch(g1, v1, sg1, j1)
      wait_fetch(g0, v0, sg0, j0)
      scale(g0, v0)
      pltpu.async_copy(g0, h_sh.at[rbuf.at[j0]], ss0, add=True)
      deg_scatter(j0)
      wait_fetch(g1, v1, sg1, j1)
      scale(g1, v1)
      pltpu.make_async_copy(g0, h_sh.at[rbuf.at[j0]], ss0).wait()

      @pl.when(jj < NCH // 2 - 1)
      def _():
        start_fetch(g0, v0, sg0, j0 + 2)

      pltpu.async_copy(g1, h_sh.at[rbuf.at[j1]], ss1, add=True)
      deg_scatter(j1)
      pltpu.make_async_copy(g1, h_sh.at[rbuf.at[j1]], ss1).wait()

    plsc.subcore_barrier()

    pltpu.sync_copy(h_sh.at[pl.ds(sid * RPT, RPT)],
                    out_hbm.at[cid, pl.ds(sid * RPT, RPT)])
    if with_deg:
      pltpu.sync_copy(deg_sh.at[pl.ds(sid * RPT, RPT)],
                      deg_hbm.at[cid, pl.ds(sid * RPT, RPT)])

  return spmm


_spmm1 = _make_spmm(D1, with_deg=True)
_spmm2 = _make_spmm(D1, with_deg=False)


def _tc_xw1(x, W1):
  def body(x_ref, w_ref, o_ref):
    o_ref[...] = jnp.dot(x_ref[...], w_ref[...],
                         preferred_element_type=jnp.float32)
  return pl.pallas_call(
      body, out_shape=jax.ShapeDtypeStruct((N, D1), jnp.float32))(x, W1)


def _tc_mid(hp, degp, b1, W2):
  def body(hp_ref, d_ref, b1_ref, w2_ref, o_ref):
    recip = 1.0 / (d_ref[0] + d_ref[1])
    h = (hp_ref[0] + hp_ref[1]) * recip + b1_ref[...]
    h = jnp.maximum(h, 0.0)
    o_ref[...] = jnp.dot(h, w2_ref[...], preferred_element_type=jnp.float32)
  return pl.pallas_call(
      body, out_shape=jax.ShapeDtypeStruct((N, D1), jnp.float32))(
          hp, degp, b1, W2)


def _tc_out(op, degp, b2):
  def body(op_ref, d_ref, b2_ref, o_ref):
    recip = 1.0 / (d_ref[0] + d_ref[1])
    z = (op_ref[0][:, :D2] + op_ref[1][:, :D2]) * recip + b2_ref[...]
    m = jnp.max(z, axis=1, keepdims=True)
    s = jnp.sum(jnp.exp(z - m), axis=1, keepdims=True)
    o_ref[...] = z - (m + jnp.log(s))
  return pl.pallas_call(
      body, out_shape=jax.ShapeDtypeStruct((N, D2), jnp.float32))(
          op, degp, b2)


def kernel(x, edge_index, edge_weights, W1, b1, W2, b2):
  rows = edge_index[0].reshape(E // CH, CH)
  cols = edge_index[1].reshape(E // CH, CH)
  ew = edge_weights.reshape(E // CH, CH)
  valx = jnp.broadcast_to(
      edge_weights.reshape(E // CH, L, CH // L, 1),
      (E // CH, L, CH // L, L)).reshape(E // CH, L, CH)

  W2p = jnp.pad(W2, ((0, 0), (0, D1 - D2)))

  xw = _tc_xw1(x, W1)
  hp, degp = _spmm1(rows, cols, ew, valx, xw)
  degp2 = jnp.reshape(degp, (NC, N, 1))
  hw = _tc_mid(hp, degp2, b1, W2p)
  op, = _spmm2(rows, cols, ew, valx, hw)
  return _tc_out(op, degp2, b2)


# speedup vs baseline: 1.0232x; 1.0232x over previous
import dataclasses
import functools

import jax
import jax.numpy as jnp
from jax import lax
from jax.experimental import pallas as pl
from jax.experimental.pallas import tpu as pltpu
from jax.experimental.pallas import tpu_sc as plsc

N = 4096
E = 131072
D1 = 128
D2 = 64

NC = 2
NS = 16
NW = NC * NS
L = 16

CH = 128
EPT = E // NW
NCH = EPT // CH
RPT = N // NS


def _make_spmm(D, with_deg):
  mesh = plsc.VectorSubcoreMesh(core_axis_name="c", subcore_axis_name="s")
  out_type = [jax.ShapeDtypeStruct((NC, N, D), jnp.float32)]
  if with_deg:
    out_type.append(jax.ShapeDtypeStruct((NC, NS, N), jnp.float32))
  scratch = [
      pltpu.VMEM((NCH, CH), jnp.int32),
      pltpu.VMEM((NCH, CH), jnp.int32),
      pltpu.VMEM((NCH, CH), jnp.float32),
      pltpu.VMEM((1, L, CH), jnp.float32),
      pltpu.VMEM((1, L, CH), jnp.float32),
      pltpu.VMEM((CH, D), jnp.float32),
      pltpu.VMEM((CH, D), jnp.float32),
      pltpu.VMEM_SHARED((N, D), jnp.float32),
      pltpu.VMEM((N,), jnp.float32),
      pltpu.SemaphoreType.DMA,
      pltpu.SemaphoreType.DMA,
      pltpu.SemaphoreType.DMA,
      pltpu.SemaphoreType.DMA,
  ]

  cp = pltpu.CompilerParams()
  if "needs_layout_passes" in pltpu.CompilerParams.__dataclass_fields__:
    cp = dataclasses.replace(cp, needs_layout_passes=False)

  @functools.partial(pl.kernel, mesh=mesh, out_type=out_type,
                     scratch_types=scratch, compiler_params=cp)
  def spmm(rows_hbm, cols_hbm, ew_hbm, valx_hbm, xw_hbm, *out_and_scratch):
    if with_deg:
      out_hbm, deg_hbm = out_and_scratch[0], out_and_scratch[1]
      rest = out_and_scratch[2:]
    else:
      out_hbm = out_and_scratch[0]
      rest = out_and_scratch[1:]
    rbuf, cbuf, wbuf, v0, v1, g0, g1, h_sh, degbuf, sg0, sg1, ss0, ss1 = rest
    cid = lax.axis_index("c")
    sid = lax.axis_index("s")
    wid = sid * NC + cid

    @plsc.parallel_loop(0, CH, unroll=4)
    def _(e):
      for c0 in range(0, D, L):
        g0[e, pl.ds(c0, L)] = jnp.zeros((L,), jnp.float32)

    for k in range(RPT // CH):
      pltpu.sync_copy(g0, h_sh.at[pl.ds(sid * RPT + k * CH, CH)])
    if with_deg:
      @plsc.parallel_loop(0, N // L, unroll=4)
      def _(i):
        degbuf[pl.ds(i * L, L)] = jnp.zeros((L,), jnp.float32)

    pltpu.sync_copy(rows_hbm.at[pl.ds(wid * NCH, NCH)], rbuf)
    pltpu.sync_copy(cols_hbm.at[pl.ds(wid * NCH, NCH)], cbuf)
    pltpu.sync_copy(ew_hbm.at[pl.ds(wid * NCH, NCH)], wbuf)

    plsc.subcore_barrier()

    def start_fetch(g, v, sg, j):
      pltpu.async_copy(xw_hbm.at[cbuf.at[j]], g, sg)
      pltpu.async_copy(valx_hbm.at[pl.ds(wid * NCH + j, 1)], v, sg)

    def wait_fetch(g, v, sg, j):
      pltpu.make_async_copy(xw_hbm.at[cbuf.at[j]], g, sg).wait()
      pltpu.make_async_copy(valx_hbm.at[pl.ds(wid * NCH + j, 1)], v, sg).wait()

    def scale(g, v):
      @plsc.parallel_loop(0, L, unroll=4)
      def _(e8):
        for jj in range(CH // L):
          e = e8 * (CH // L) + jj
          vv = v[0, e8, pl.ds(jj * L, L)]
          for c0 in range(0, D, L):
            g[e, pl.ds(c0, L)] = g[e, pl.ds(c0, L)] * vv

    def deg_scatter(j):
      if with_deg:
        for k in range(CH // L):
          plsc.addupdate_scatter(degbuf, [rbuf[j, pl.ds(k * L, L)]],
                                 wbuf[j, pl.ds(k * L, L)])

    start_fetch(g0, v0, sg0, 0)

    @pl.loop(0, NCH // 2)
    def _(jj):
      j0 = jj * 2
      j1 = j0 + 1

      @pl.when(jj > 0)
      def _():
        pltpu.make_async_copy(g1, h_sh.at[rbuf.at[j0 - 1]], ss1).wait()

      start_fetch(g1, v1, sg1, j1)
      wait_fetch(g0, v0, sg0, j0)
      scale(g0, v0)
      pltpu.async_copy(g0, h_sh.at[rbuf.at[j0]], ss0, add=True)
      deg_scatter(j0)
      wait_fetch(g1, v1, sg1, j1)
      scale(g1, v1)
      pltpu.make_async_copy(g0, h_sh.at[rbuf.at[j0]], ss0).wait()

      @pl.when(jj < NCH // 2 - 1)
      def _():
        start_fetch(g0, v0, sg0, j0 + 2)

      pltpu.async_copy(g1, h_sh.at[rbuf.at[j1]], ss1, add=True)
      deg_scatter(j1)

    pltpu.make_async_copy(g1, h_sh.at[rbuf.at[NCH - 1]], ss1).wait()
    if with_deg:
      pltpu.sync_copy(degbuf, deg_hbm.at[cid, sid])

    plsc.subcore_barrier()

    pltpu.sync_copy(h_sh.at[pl.ds(sid * RPT, RPT)],
                    out_hbm.at[cid, pl.ds(sid * RPT, RPT)])

  return spmm


_spmm1 = _make_spmm(D1, with_deg=True)
_spmm2 = _make_spmm(D1, with_deg=False)


def _tc_xw1(x, W1):
  def body(x_ref, w_ref, o_ref):
    o_ref[...] = jnp.dot(x_ref[...], w_ref[...],
                         preferred_element_type=jnp.float32)
  return pl.pallas_call(
      body, out_shape=jax.ShapeDtypeStruct((N, D1), jnp.float32))(x, W1)


def _tc_mid(hp, degp, b1, W2):
  def body(hp_ref, d_ref, b1_ref, w2_ref, o_ref, r_ref):
    recip = 1.0 / jnp.sum(d_ref[...], axis=1, keepdims=True)
    r_ref[...] = recip
    h = (hp_ref[0] + hp_ref[1]) * recip + b1_ref[...]
    h = jnp.maximum(h, 0.0)
    o_ref[...] = jnp.dot(h, w2_ref[...], preferred_element_type=jnp.float32)
  return pl.pallas_call(
      body, out_shape=[jax.ShapeDtypeStruct((N, D1), jnp.float32),
                       jax.ShapeDtypeStruct((N, 1), jnp.float32)])(
          hp, degp, b1, W2)


def _tc_out(op, recip, b2):
  def body(op_ref, r_ref, b2_ref, o_ref):
    z = (op_ref[0][:, :D2] + op_ref[1][:, :D2]) * r_ref[...] + b2_ref[...]
    m = jnp.max(z, axis=1, keepdims=True)
    s = jnp.sum(jnp.exp(z - m), axis=1, keepdims=True)
    o_ref[...] = z - (m + jnp.log(s))
  return pl.pallas_call(
      body, out_shape=jax.ShapeDtypeStruct((N, D2), jnp.float32))(
          op, recip, b2)


def kernel(x, edge_index, edge_weights, W1, b1, W2, b2):
  rows = edge_index[0].reshape(E // CH, CH)
  cols = edge_index[1].reshape(E // CH, CH)
  ew = edge_weights.reshape(E // CH, CH)
  valx = jnp.broadcast_to(
      edge_weights.reshape(E // CH, L, CH // L, 1),
      (E // CH, L, CH // L, L)).reshape(E // CH, L, CH)

  W2p = jnp.pad(W2, ((0, 0), (0, D1 - D2)))

  xw = _tc_xw1(x, W1)
  hp, degp = _spmm1(rows, cols, ew, valx, xw)
  degp2 = jnp.transpose(jnp.reshape(degp, (NC * NS, N)))
  hw, recip = _tc_mid(hp, degp2, b1, W2p)
  op, = _spmm2(rows, cols, ew, valx, hw)
  return _tc_out(op, recip, b2)
